# Initial kernel scaffold; baseline (speedup 1.0000x reference)
import jax
import jax.numpy as jnp
from jax.experimental import pallas as pl


def kernel(x, edge_index, batch, W1, b1, g1, be1, m1, v1, W2, b2, W3, b3, g2, be2, m2, v2, W4, b4, Wl, bl):
    raise NotImplementedError("write your pallas kernel here")


# speedup vs baseline: 3.5781x; 3.5781x over previous
import functools

import jax
import jax.numpy as jnp
from jax import lax
from jax.experimental import pallas as pl
from jax.experimental.pallas import tpu as pltpu
from jax.experimental.pallas import tpu_sc as plsc

N = 50000
E = 800000
F_IN = 33
H = 64
C = 6
G = 512

NC = 2
NS = 16
LANES = 16

BN = 512
NP = 50176
NBLK = NP // BN
N2 = NP // 2
ROWS_PER_TILE = N2 // NS
ACC_ROWS = N2 + 512
ZROWS_PER_TILE = ACC_ROWS // NS
EP = 800768
EGROUP = 128
NGROUPS = EP // EGROUP
GPT = NGROUPS // NS

DIN1 = 48


def _make_agg(dp):
  mesh = plsc.VectorSubcoreMesh(
      core_axis_name="c", subcore_axis_name="s", num_cores=NC, num_subcores=NS)

  @functools.partial(
      pl.kernel,
      out_type=jax.ShapeDtypeStruct((NP, dp), jnp.float32),
      mesh=mesh,
      compiler_params=pltpu.CompilerParams(use_tc_tiling_on_sc=False),
      scratch_types=[
          pltpu.VMEM((EGROUP,), jnp.int32),
          pltpu.VMEM((EGROUP,), jnp.int32),
          pltpu.VMEM((EGROUP,), jnp.int32),
          pltpu.VMEM((EGROUP, dp), jnp.float32),
          pltpu.VMEM((200, dp), jnp.float32),
          pltpu.VMEM_SHARED((ACC_ROWS, dp), jnp.float32),
          pltpu.SemaphoreType.DMA,
      ],
  )
  def agg(table_hbm, src_hbm, dst_hbm, out_hbm, src_v, dst_v, ldst_v, rows_v,
          zbuf, acc, sem):
    cid = lax.axis_index("c")
    sid = lax.axis_index("s")
    base = cid * N2

    zeros16 = jnp.zeros((LANES,), jnp.float32)

    def zrow(r, carry):
      for k in range(dp // LANES):
        zbuf[r, pl.ds(k * LANES, LANES)] = zeros16
      return carry

    lax.fori_loop(0, 200, zrow, 0)

    def zcp(i, carry):
      pltpu.sync_copy(zbuf, acc.at[pl.ds(sid * ZROWS_PER_TILE + i * 200, 200)])
      return carry

    lax.fori_loop(0, ZROWS_PER_TILE // 200, zcp, 0)
    plsc.subcore_barrier()

    def body(g, carry):
      goff = pl.multiple_of((sid * GPT + g) * EGROUP, EGROUP)
      pltpu.sync_copy(src_hbm.at[pl.ds(goff, EGROUP)], src_v)
      pltpu.sync_copy(dst_hbm.at[pl.ds(goff, EGROUP)], dst_v)
      for k in range(EGROUP // LANES):
        d = dst_v[pl.ds(k * LANES, LANES)]
        ld = d - base
        ok = (ld >= 0) & (ld < N2)
        ldst_v[pl.ds(k * LANES, LANES)] = jnp.where(ok, ld, N2)
      pltpu.async_copy(table_hbm.at[src_v], rows_v, sem).wait()
      pltpu.sync_copy(rows_v, acc.at[ldst_v], add=True)
      return carry

    lax.fori_loop(0, GPT, body, 0)
    plsc.subcore_barrier()

    pltpu.sync_copy(
        acc.at[pl.ds(sid * ROWS_PER_TILE, ROWS_PER_TILE)],
        out_hbm.at[pl.ds(base + sid * ROWS_PER_TILE, ROWS_PER_TILE)])

  return agg


_agg48 = _make_agg(DIN1)
_agg64 = _make_agg(H)


def _mlp_body(x_ref, a_ref, b_ref, wa_ref, ba_ref, g_ref, be_ref, m_ref,
              v_ref, wb_ref, bb_ref, h_ref, p_ref):
  z = x_ref[...] + a_ref[...]
  h = jnp.dot(z, wa_ref[...], preferred_element_type=jnp.float32) + ba_ref[...]
  scale = g_ref[...] * lax.rsqrt(v_ref[...] + 1e-5)
  h = scale * (h - m_ref[...]) + be_ref[...]
  h = jnp.maximum(h, 0.0)
  h = jnp.dot(h, wb_ref[...], preferred_element_type=jnp.float32) + bb_ref[...]
  h = jnp.maximum(h, 0.0)
  h_ref[...] = h
  bvals = b_ref[0, 0, :]
  ids = lax.broadcasted_iota(jnp.int32, (G, BN), 0)
  onehot = (ids == bvals[None, :]).astype(jnp.float32)
  pp = jnp.dot(onehot, h, preferred_element_type=jnp.float32)

  @pl.when(pl.program_id(0) == 0)
  def _():
    p_ref[...] = pp

  @pl.when(pl.program_id(0) != 0)
  def _():
    p_ref[...] += pp


def _make_mlp(din):
  full = lambda i: (0, 0)
  return pl.pallas_call(
      _mlp_body,
      grid=(NBLK,),
      in_specs=[
          pl.BlockSpec((BN, din), lambda i: (i, 0)),
          pl.BlockSpec((BN, din), lambda i: (i, 0)),
          pl.BlockSpec((1, 1, BN), lambda i: (i, 0, 0)),
          pl.BlockSpec((din, H), full),
          pl.BlockSpec((1, H), full),
          pl.BlockSpec((1, H), full),
          pl.BlockSpec((1, H), full),
          pl.BlockSpec((1, H), full),
          pl.BlockSpec((1, H), full),
          pl.BlockSpec((H, H), full),
          pl.BlockSpec((1, H), full),
      ],
      out_specs=[
          pl.BlockSpec((BN, H), lambda i: (i, 0)),
          pl.BlockSpec((G, H), full),
      ],
      out_shape=[
          jax.ShapeDtypeStruct((NP, H), jnp.float32),
          jax.ShapeDtypeStruct((G, H), jnp.float32),
      ],
  )


_mlp48 = _make_mlp(DIN1)
_mlp64 = _make_mlp(H)


def _final_body(p1_ref, p2_ref, wl_ref, bl_ref, o_ref):
  o = jnp.dot(p1_ref[...], wl_ref[:H, :], preferred_element_type=jnp.float32)
  o += jnp.dot(p2_ref[...], wl_ref[H:, :], preferred_element_type=jnp.float32)
  o_ref[...] = o + bl_ref[...]


_final = pl.pallas_call(
    _final_body,
    out_shape=jax.ShapeDtypeStruct((G, C), jnp.float32),
)


@jax.jit
def _impl(x, edge_index, batch, W1, b1, g1, be1, m1, v1, W2, b2, W3, b3, g2,
          be2, m2, v2, W4, b4, Wl, bl):
  src = edge_index[0]
  dst = edge_index[1]
  srcp = jnp.concatenate([src, jnp.zeros((EP - E,), jnp.int32)])
  dstp = jnp.concatenate([dst, jnp.full((EP - E,), NP, jnp.int32)])
  xp = jnp.zeros((NP, DIN1), jnp.float32).at[:N, :F_IN].set(x)
  W1p = jnp.zeros((DIN1, H), jnp.float32).at[:F_IN].set(W1)
  batchp = jnp.concatenate(
      [batch, jnp.full((NP - N,), -1, jnp.int32)]).reshape(NBLK, 1, BN)
  r = lambda a: a.reshape(1, H)

  agg1 = _agg48(xp, srcp, dstp)
  h1, p1 = _mlp48(xp, agg1, batchp, W1p, r(b1), r(g1), r(be1), r(m1), r(v1),
                  W2, r(b2))
  agg2 = _agg64(h1, srcp, dstp)
  _, p2 = _mlp64(h1, agg2, batchp, W3, r(b3), r(g2), r(be2), r(m2), r(v2),
                 W4, r(b4))
  return _final(p1, p2, Wl, bl.reshape(1, C))


def kernel(x, edge_index, batch, W1, b1, g1, be1, m1, v1, W2, b2, W3, b3, g2,
           be2, m2, v2, W4, b4, Wl, bl):
  return _impl(x, edge_index, batch, W1, b1, g1, be1, m1, v1, W2, b2, W3, b3,
               g2, be2, m2, v2, W4, b4, Wl, bl)


# speedup vs baseline: 3.6856x; 1.0300x over previous
import functools

import jax
import jax.numpy as jnp
from jax import lax
from jax.experimental import pallas as pl
from jax.experimental.pallas import tpu as pltpu
from jax.experimental.pallas import tpu_sc as plsc

N = 50000
E = 800000
F_IN = 33
H = 64
C = 6
G = 512

NC = 2
NS = 16
LANES = 16

BN = 512
NP = 50176
NBLK = NP // BN
N2 = NP // 2
ROWS_PER_TILE = N2 // NS
ACC_ROWS = N2 + 16
ZROWS = ACC_ROWS // NS

EGROUP = 128
SCH = 4
NSC = 98
GPT = NSC * SCH
NGROUPS = GPT * NS
NGROUPS_ALLOC = NGROUPS + SCH
EP = NGROUPS_ALLOC * EGROUP

DIN1 = 48
HH = 32


def _make_agg(dp):
  mesh = plsc.VectorSubcoreMesh(
      core_axis_name="c", subcore_axis_name="s", num_cores=NC, num_subcores=NS)

  @functools.partial(
      pl.kernel,
      out_type=jax.ShapeDtypeStruct((NP, dp), jnp.float32),
      mesh=mesh,
      compiler_params=pltpu.CompilerParams(use_tc_tiling_on_sc=False),
      scratch_types=[
          pltpu.VMEM((2, SCH, 2, EGROUP), jnp.int32),
          pltpu.VMEM((2, SCH, EGROUP), jnp.int32),
          pltpu.VMEM((2, SCH, EGROUP, dp), jnp.float32),
          pltpu.VMEM_SHARED((ACC_ROWS, dp), jnp.float32),
          pltpu.SemaphoreType.DMA,
          pltpu.SemaphoreType.DMA,
          pltpu.SemaphoreType.DMA,
          pltpu.SemaphoreType.DMA,
      ],
  )
  def agg(table_hbm, idx2_hbm, zeros_hbm, out_hbm, idx_v, ldst_v, rows_v,
          acc, semg0, semg1, sems0, sems1):
    cid = lax.axis_index("c")
    sid = lax.axis_index("s")
    base = cid * N2
    semg = (semg0, semg1)
    sems = (sems0, sems1)

    pltpu.sync_copy(zeros_hbm, acc.at[pl.ds(sid * ZROWS, ZROWS)])
    plsc.subcore_barrier()

    def load_idx(c, p):
      grow = sid * GPT + c * SCH
      pltpu.sync_copy(idx2_hbm.at[pl.ds(grow, SCH)], idx_v.at[p])

    def compute_ldst(p):
      for j in range(SCH):
        for k in range(EGROUP // LANES):
          d = idx_v[p, j, 1, pl.ds(k * LANES, LANES)]
          ld = d - base
          ok = (ld >= 0) & (ld < N2)
          ldst_v[p, j, pl.ds(k * LANES, LANES)] = jnp.where(ok, ld, N2)

    def fire_gathers(p):
      for j in range(SCH):
        pltpu.async_copy(table_hbm.at[idx_v.at[p, j, 0]], rows_v.at[p, j],
                         semg[p])

    def drain_gathers(p):
      for j in range(SCH):
        pltpu.make_async_copy(table_hbm.at[idx_v.at[p, j, 0]],
                              rows_v.at[p, j], semg[p]).wait()

    def fire_scatters(p):
      for j in range(SCH):
        pltpu.async_copy(rows_v.at[p, j], acc.at[ldst_v.at[p, j]], sems[p],
                         add=True)

    def drain_scatters(p):
      for j in range(SCH):
        pltpu.make_async_copy(rows_v.at[p, j], acc.at[ldst_v.at[p, j]],
                              sems[p]).wait()

    trash16 = jnp.full((LANES,), N2, jnp.int32)
    for j in range(SCH):
      for k in range(EGROUP // LANES):
        ldst_v[1, j, pl.ds(k * LANES, LANES)] = trash16
    fire_scatters(1)
    load_idx(0, 0)
    compute_ldst(0)
    fire_gathers(0)

    def body(i, carry):
      for p in range(2):
        c = 2 * i + p
        drain_scatters(1 - p)
        load_idx(c + 1, 1 - p)
        compute_ldst(1 - p)
        drain_gathers(p)
        fire_gathers(1 - p)
        fire_scatters(p)
      return carry

    lax.fori_loop(0, NSC // 2, body, 0)
    drain_scatters(1)
    drain_gathers(0)
    plsc.subcore_barrier()

    pltpu.sync_copy(
        acc.at[pl.ds(sid * ROWS_PER_TILE, ROWS_PER_TILE)],
        out_hbm.at[pl.ds(base + sid * ROWS_PER_TILE, ROWS_PER_TILE)])

  return agg


_agg48 = _make_agg(DIN1)
_agg32 = _make_agg(HH)


def _mlp_body(split_h, x_refs, a_refs, b_ref, wa_ref, ba_ref, g_ref, be_ref,
              m_ref, v_ref, wb_ref, bb_ref, *out_refs):
  z = jnp.concatenate([r[...] for r in x_refs], axis=1) + (
      jnp.concatenate([r[...] for r in a_refs], axis=1))
  h = jnp.dot(z, wa_ref[...], preferred_element_type=jnp.float32) + ba_ref[...]
  scale = g_ref[...] * lax.rsqrt(v_ref[...] + 1e-5)
  h = scale * (h - m_ref[...]) + be_ref[...]
  h = jnp.maximum(h, 0.0)
  h = jnp.dot(h, wb_ref[...], preferred_element_type=jnp.float32) + bb_ref[...]
  h = jnp.maximum(h, 0.0)
  if split_h:
    out_refs[0][...] = h[:, :HH]
    out_refs[1][...] = h[:, HH:]
  p_ref = out_refs[-1]
  bvals = b_ref[0, 0, :]
  ids = lax.broadcasted_iota(jnp.int32, (G, BN), 0)
  onehot = (ids == bvals[None, :]).astype(jnp.float32)
  pp = jnp.dot(onehot, h, preferred_element_type=jnp.float32)

  @pl.when(pl.program_id(0) == 0)
  def _():
    p_ref[...] = pp

  @pl.when(pl.program_id(0) != 0)
  def _():
    p_ref[...] += pp


_FULL = lambda i: (0, 0)
_ROWB = lambda d: pl.BlockSpec((BN, d), lambda i: (i, 0))


def _mlp_specs(din_parts):
  specs = [_ROWB(d) for d in din_parts]
  specs += [_ROWB(d) for d in din_parts]
  specs += [pl.BlockSpec((1, 1, BN), lambda i: (i, 0, 0))]
  din = sum(din_parts)
  specs += [pl.BlockSpec((din, H), _FULL)]
  specs += [pl.BlockSpec((1, H), _FULL)] * 5
  specs += [pl.BlockSpec((H, H), _FULL)]
  specs += [pl.BlockSpec((1, H), _FULL)]
  return specs


def _wrap_mlp(split_h, nparts):
  def body(*refs):
    x_refs = refs[:nparts]
    a_refs = refs[nparts:2 * nparts]
    rest = refs[2 * nparts:]
    _mlp_body(split_h, x_refs, a_refs, *rest)
  return body


_mlp1 = pl.pallas_call(
    _wrap_mlp(True, 1),
    grid=(NBLK,),
    in_specs=_mlp_specs([DIN1]),
    out_specs=[_ROWB(HH), _ROWB(HH), pl.BlockSpec((G, H), _FULL)],
    out_shape=[
        jax.ShapeDtypeStruct((NP, HH), jnp.float32),
        jax.ShapeDtypeStruct((NP, HH), jnp.float32),
        jax.ShapeDtypeStruct((G, H), jnp.float32),
    ],
)

_mlp2 = pl.pallas_call(
    _wrap_mlp(False, 2),
    grid=(NBLK,),
    in_specs=_mlp_specs([HH, HH]),
    out_specs=[pl.BlockSpec((G, H), _FULL)],
    out_shape=[jax.ShapeDtypeStruct((G, H), jnp.float32)],
)


def _final_body(p1_ref, p2_ref, wl_ref, bl_ref, o_ref):
  o = jnp.dot(p1_ref[...], wl_ref[:H, :], preferred_element_type=jnp.float32)
  o += jnp.dot(p2_ref[...], wl_ref[H:, :], preferred_element_type=jnp.float32)
  o_ref[...] = o + bl_ref[...]


_final = pl.pallas_call(
    _final_body,
    out_shape=jax.ShapeDtypeStruct((G, C), jnp.float32),
)


@jax.jit
def _impl(x, edge_index, batch, W1, b1, g1, be1, m1, v1, W2, b2, W3, b3, g2,
          be2, m2, v2, W4, b4, Wl, bl):
  src = edge_index[0]
  dst = edge_index[1]
  srcp = jnp.concatenate([src, jnp.zeros((EP - E,), jnp.int32)]).reshape(
      NGROUPS_ALLOC, EGROUP)
  dstp = jnp.concatenate([dst, jnp.full((EP - E,), NP, jnp.int32)]).reshape(
      NGROUPS_ALLOC, EGROUP)
  idx2 = jnp.stack([srcp, dstp], axis=1)
  xp = jnp.zeros((NP, DIN1), jnp.float32).at[:N, :F_IN].set(x)
  W1p = jnp.zeros((DIN1, H), jnp.float32).at[:F_IN].set(W1)
  batchp = jnp.concatenate(
      [batch, jnp.full((NP - N,), -1, jnp.int32)]).reshape(NBLK, 1, BN)
  z48 = jnp.zeros((ZROWS, DIN1), jnp.float32)
  z32 = jnp.zeros((ZROWS, HH), jnp.float32)
  r = lambda a: a.reshape(1, H)

  agg1 = _agg48(xp, idx2, z48)
  h1a, h1b, p1 = _mlp1(xp, agg1, batchp, W1p, r(b1), r(g1), r(be1), r(m1),
                       r(v1), W2, r(b2))
  agg2a = _agg32(h1a, idx2, z32)
  agg2b = _agg32(h1b, idx2, z32)
  (p2,) = _mlp2(h1a, h1b, agg2a, agg2b, batchp, W3, r(b3), r(g2), r(be2),
                r(m2), r(v2), W4, r(b4))
  return _final(p1, p2, Wl, bl.reshape(1, C))


def kernel(x, edge_index, batch, W1, b1, g1, be1, m1, v1, W2, b2, W3, b3, g2,
           be2, m2, v2, W4, b4, Wl, bl):
  return _impl(x, edge_index, batch, W1, b1, g1, be1, m1, v1, W2, b2, W3, b3,
               g2, be2, m2, v2, W4, b4, Wl, bl)


# speedup vs baseline: 4.8254x; 1.3093x over previous
import functools

import jax
import jax.numpy as jnp
from jax import lax
from jax.experimental import pallas as pl
from jax.experimental.pallas import tpu as pltpu
from jax.experimental.pallas import tpu_sc as plsc

N = 50000
E = 800000
F_IN = 33
H = 64
C = 6
G = 512

NC = 2
NS = 16
LANES = 16

BN = 512
NP = 50176
NBLK = NP // BN
N2 = NP // 2
ROWS_PER_TILE = N2 // NS
ACC_ROWS = N2 + 16
ZROWS = ACC_ROWS // NS

EGROUP = 128
GPT = 392
NGROUPS = GPT * NS
NGROUPS_ALLOC = NGROUPS + 4
EP = NGROUPS_ALLOC * EGROUP

DIN1 = 48
HH = 32


def _make_agg(dp, sch):
  nsc = GPT // sch
  assert nsc * sch == GPT and nsc % 2 == 0
  mesh = plsc.VectorSubcoreMesh(
      core_axis_name="c", subcore_axis_name="s", num_cores=NC, num_subcores=NS)

  @functools.partial(
      pl.kernel,
      out_type=jax.ShapeDtypeStruct((NP, dp), jnp.float32),
      mesh=mesh,
      compiler_params=pltpu.CompilerParams(use_tc_tiling_on_sc=False),
      scratch_types=[
          pltpu.VMEM((2, sch, 2, EGROUP), jnp.int32),
          pltpu.VMEM((2, sch, EGROUP), jnp.int32),
          pltpu.VMEM((2, sch, EGROUP, dp), jnp.float32),
          pltpu.VMEM_SHARED((ACC_ROWS, dp), jnp.float32),
          pltpu.SemaphoreType.DMA,
          pltpu.SemaphoreType.DMA,
          pltpu.SemaphoreType.DMA,
          pltpu.SemaphoreType.DMA,
      ],
  )
  def agg(table_hbm, idx2_hbm, zeros_hbm, out_hbm, idx_v, ldst_v, rows_v,
          acc, semg0, semg1, sems0, sems1):
    cid = lax.axis_index("c")
    sid = lax.axis_index("s")
    base = cid * N2
    semg = (semg0, semg1)
    sems = (sems0, sems1)

    pltpu.sync_copy(zeros_hbm, acc.at[pl.ds(sid * ZROWS, ZROWS)])
    plsc.subcore_barrier()

    def load_idx(c, p):
      grow = sid * GPT + c * sch
      pltpu.sync_copy(idx2_hbm.at[pl.ds(grow, sch)], idx_v.at[p])

    def compute_ldst(p):
      for j in range(sch):
        for k in range(EGROUP // LANES):
          d = idx_v[p, j, 1, pl.ds(k * LANES, LANES)]
          ld = d - base
          ok = (ld >= 0) & (ld < N2)
          ldst_v[p, j, pl.ds(k * LANES, LANES)] = jnp.where(ok, ld, N2)

    def fire_gathers(p):
      for j in range(sch):
        pltpu.async_copy(table_hbm.at[idx_v.at[p, j, 0]], rows_v.at[p, j],
                         semg[p])

    def drain_gathers(p):
      for j in range(sch):
        pltpu.make_async_copy(table_hbm.at[idx_v.at[p, j, 0]],
                              rows_v.at[p, j], semg[p]).wait()

    def fire_scatters(p):
      for j in range(sch):
        pltpu.async_copy(rows_v.at[p, j], acc.at[ldst_v.at[p, j]], sems[p],
                         add=True)

    def drain_scatters(p):
      for j in range(sch):
        pltpu.make_async_copy(rows_v.at[p, j], acc.at[ldst_v.at[p, j]],
                              sems[p]).wait()

    trash16 = jnp.full((LANES,), N2, jnp.int32)
    for j in range(sch):
      for k in range(EGROUP // LANES):
        ldst_v[1, j, pl.ds(k * LANES, LANES)] = trash16
    fire_scatters(1)
    load_idx(0, 0)
    compute_ldst(0)
    fire_gathers(0)

    def body(i, carry):
      for p in range(2):
        c = 2 * i + p
        drain_scatters(1 - p)
        load_idx(c + 1, 1 - p)
        compute_ldst(1 - p)
        drain_gathers(p)
        fire_gathers(1 - p)
        fire_scatters(p)
      return carry

    lax.fori_loop(0, nsc // 2, body, 0)
    drain_scatters(1)
    drain_gathers(0)
    plsc.subcore_barrier()

    pltpu.sync_copy(
        acc.at[pl.ds(sid * ROWS_PER_TILE, ROWS_PER_TILE)],
        out_hbm.at[pl.ds(base + sid * ROWS_PER_TILE, ROWS_PER_TILE)])

  return agg


_agg48 = _make_agg(DIN1, 4)
_agg64 = _make_agg(H, 1)


def _mlp_body(split_h, x_refs, a_refs, b_ref, wa_ref, ba_ref, g_ref, be_ref,
              m_ref, v_ref, wb_ref, bb_ref, *out_refs):
  z = jnp.concatenate([r[...] for r in x_refs], axis=1) + (
      jnp.concatenate([r[...] for r in a_refs], axis=1))
  h = jnp.dot(z, wa_ref[...], preferred_element_type=jnp.float32) + ba_ref[...]
  scale = g_ref[...] * lax.rsqrt(v_ref[...] + 1e-5)
  h = scale * (h - m_ref[...]) + be_ref[...]
  h = jnp.maximum(h, 0.0)
  h = jnp.dot(h, wb_ref[...], preferred_element_type=jnp.float32) + bb_ref[...]
  h = jnp.maximum(h, 0.0)
  if split_h:
    out_refs[0][...] = h
  p_ref = out_refs[-1]
  bvals = b_ref[0, 0, :]
  ids = lax.broadcasted_iota(jnp.int32, (G, BN), 0)
  onehot = (ids == bvals[None, :]).astype(jnp.float32)
  pp = jnp.dot(onehot, h, preferred_element_type=jnp.float32)

  @pl.when(pl.program_id(0) == 0)
  def _():
    p_ref[...] = pp

  @pl.when(pl.program_id(0) != 0)
  def _():
    p_ref[...] += pp


_FULL = lambda i: (0, 0)
_ROWB = lambda d: pl.BlockSpec((BN, d), lambda i: (i, 0))


def _mlp_specs(din_parts):
  specs = [_ROWB(d) for d in din_parts]
  specs += [_ROWB(d) for d in din_parts]
  specs += [pl.BlockSpec((1, 1, BN), lambda i: (i, 0, 0))]
  din = sum(din_parts)
  specs += [pl.BlockSpec((din, H), _FULL)]
  specs += [pl.BlockSpec((1, H), _FULL)] * 5
  specs += [pl.BlockSpec((H, H), _FULL)]
  specs += [pl.BlockSpec((1, H), _FULL)]
  return specs


def _wrap_mlp(split_h, nparts):
  def body(*refs):
    x_refs = refs[:nparts]
    a_refs = refs[nparts:2 * nparts]
    rest = refs[2 * nparts:]
    _mlp_body(split_h, x_refs, a_refs, *rest)
  return body


_mlp1 = pl.pallas_call(
    _wrap_mlp(True, 1),
    grid=(NBLK,),
    in_specs=_mlp_specs([DIN1]),
    out_specs=[_ROWB(H), pl.BlockSpec((G, H), _FULL)],
    out_shape=[
        jax.ShapeDtypeStruct((NP, H), jnp.float32),
        jax.ShapeDtypeStruct((G, H), jnp.float32),
    ],
)

_mlp2 = pl.pallas_call(
    _wrap_mlp(False, 1),
    grid=(NBLK,),
    in_specs=_mlp_specs([H]),
    out_specs=[pl.BlockSpec((G, H), _FULL)],
    out_shape=[jax.ShapeDtypeStruct((G, H), jnp.float32)],
)


def _final_body(p1_ref, p2_ref, wl_ref, bl_ref, o_ref):
  o = jnp.dot(p1_ref[...], wl_ref[:H, :], preferred_element_type=jnp.float32)
  o += jnp.dot(p2_ref[...], wl_ref[H:, :], preferred_element_type=jnp.float32)
  o_ref[...] = o + bl_ref[...]


_final = pl.pallas_call(
    _final_body,
    out_shape=jax.ShapeDtypeStruct((G, C), jnp.float32),
)


@jax.jit
def _impl(x, edge_index, batch, W1, b1, g1, be1, m1, v1, W2, b2, W3, b3, g2,
          be2, m2, v2, W4, b4, Wl, bl):
  src = edge_index[0]
  dst = edge_index[1]
  srcp = jnp.concatenate([src, jnp.zeros((EP - E,), jnp.int32)]).reshape(
      NGROUPS_ALLOC, EGROUP)
  dstp = jnp.concatenate([dst, jnp.full((EP - E,), NP, jnp.int32)]).reshape(
      NGROUPS_ALLOC, EGROUP)
  idx2 = jnp.stack([srcp, dstp], axis=1)
  xp = jnp.zeros((NP, DIN1), jnp.float32).at[:N, :F_IN].set(x)
  W1p = jnp.zeros((DIN1, H), jnp.float32).at[:F_IN].set(W1)
  batchp = jnp.concatenate(
      [batch, jnp.full((NP - N,), -1, jnp.int32)]).reshape(NBLK, 1, BN)
  z48 = jnp.zeros((ZROWS, DIN1), jnp.float32)
  z64 = jnp.zeros((ZROWS, H), jnp.float32)
  r = lambda a: a.reshape(1, H)

  agg1 = _agg48(xp, idx2, z48)
  h1, p1 = _mlp1(xp, agg1, batchp, W1p, r(b1), r(g1), r(be1), r(m1),
                 r(v1), W2, r(b2))
  agg2 = _agg64(h1, idx2, z64)
  (p2,) = _mlp2(h1, agg2, batchp, W3, r(b3), r(g2), r(be2),
                r(m2), r(v2), W4, r(b4))
  return _final(p1, p2, Wl, bl.reshape(1, C))


def kernel(x, edge_index, batch, W1, b1, g1, be1, m1, v1, W2, b2, W3, b3, g2,
           be2, m2, v2, W4, b4, Wl, bl):
  return _impl(x, edge_index, batch, W1, b1, g1, be1, m1, v1, W2, b2, W3, b3,
               g2, be2, m2, v2, W4, b4, Wl, bl)
